# Initial kernel scaffold; baseline (speedup 1.0000x reference)
import jax
import jax.numpy as jnp
from jax.experimental import pallas as pl


def kernel(x, Wg, W1, b1, W2, b2):
    raise NotImplementedError("write your pallas kernel here")


# speedup vs baseline: 1.5935x; 1.5935x over previous
import functools
import jax
import jax.numpy as jnp
from jax.experimental import pallas as pl
from jax.experimental.pallas import tpu as pltpu

T = 4096
D = 1024
H = 1024
E = 8
K = 2

BT = 512


def _moe_block(x_ref, wg_ref, w1_ref, b1_ref, w2_ref, b2_ref, y_ref):
    e = pl.program_id(1)
    x = x_ref[...]
    logits = jnp.dot(x, wg_ref[...])
    ids = jax.lax.broadcasted_iota(jnp.int32, logits.shape, 1)
    m1 = jnp.max(logits, axis=1, keepdims=True)
    i1 = jnp.min(jnp.where(logits == m1, ids, E), axis=1, keepdims=True)
    masked = jnp.where(ids == i1, -jnp.inf, logits)
    m2 = jnp.max(masked, axis=1, keepdims=True)
    i2 = jnp.min(jnp.where(masked == m2, ids, E), axis=1, keepdims=True)
    s = jnp.exp(m2 - m1)
    w1g = 1.0 / (1.0 + s)
    w2g = 1.0 - w1g
    ge = jnp.where(i1 == e, w1g, 0.0) + jnp.where(i2 == e, w2g, 0.0)

    h = jax.nn.gelu(jnp.dot(x, w1_ref[0]) + b1_ref[0])
    eo = jnp.dot(h, w2_ref[0]) + b2_ref[0]

    @pl.when(e == 0)
    def _():
        y_ref[...] = ge * eo

    @pl.when(e != 0)
    def _():
        y_ref[...] += ge * eo


def kernel(x, Wg, W1, b1, W2, b2):
    grid = (T // BT, E)
    return pl.pallas_call(
        _moe_block,
        grid=grid,
        in_specs=[
            pl.BlockSpec((BT, D), lambda i, e: (i, 0)),
            pl.BlockSpec((D, E), lambda i, e: (0, 0)),
            pl.BlockSpec((1, D, H), lambda i, e: (e, 0, 0)),
            pl.BlockSpec((1, 1, H), lambda i, e: (e, 0, 0)),
            pl.BlockSpec((1, H, D), lambda i, e: (e, 0, 0)),
            pl.BlockSpec((1, 1, D), lambda i, e: (e, 0, 0)),
        ],
        out_specs=pl.BlockSpec((BT, D), lambda i, e: (i, 0)),
        out_shape=jax.ShapeDtypeStruct((T, D), jnp.float32),
        compiler_params=pltpu.CompilerParams(
            dimension_semantics=("parallel", "arbitrary"),
        ),
    )(x, Wg, W1, b1.reshape(E, 1, H), W2, b2.reshape(E, 1, D))
